# Initial kernel scaffold; baseline (speedup 1.0000x reference)
import jax
import jax.numpy as jnp
from jax.experimental import pallas as pl


def kernel(x, bn_gamma, bn_beta, W1, att_src1, att_dst1, b1, W2, att_src2, att_dst2, b2, Wp, bp, edge_index, target):
    raise NotImplementedError("write your pallas kernel here")


# speedup vs baseline: 3.1059x; 3.1059x over previous
import functools

import jax
import jax.numpy as jnp
from jax import lax
from jax.experimental import pallas as pl
from jax.experimental.pallas import tpu as pltpu
from jax.experimental.pallas import tpu_sc as plsc

N = 10000
NP = 10016
E_TOT = 170000
EP = 170496
EPT3 = EP // 32
EPT4 = EP // 16
BM = 400
GRID_M = 25
C = 768
CG = C // 16
NCHUNK = 8
RCH = 1280
RPT = RCH // 16
CAP = 1536
CPT = 48


def _stats_body(x_ref, s1_ref, s2_ref):
    i = pl.program_id(0)
    xb = x_ref[...]
    s1 = jnp.sum(xb, axis=0, keepdims=True)
    s2 = jnp.sum(xb * xb, axis=0, keepdims=True)

    @pl.when(i == 0)
    def _():
        s1_ref[...] = s1
        s2_ref[...] = s2

    @pl.when(i > 0)
    def _():
        s1_ref[...] += s1
        s2_ref[...] += s2


def _stats(x):
    d = x.shape[1]
    return pl.pallas_call(
        _stats_body,
        grid=(GRID_M,),
        in_specs=[pl.BlockSpec((BM, d), lambda i: (i, 0))],
        out_specs=[pl.BlockSpec((1, d), lambda i: (0, 0)),
                   pl.BlockSpec((1, d), lambda i: (0, 0))],
        out_shape=[jax.ShapeDtypeStruct((1, d), jnp.float32),
                   jax.ShapeDtypeStruct((1, d), jnp.float32)],
    )(x)


def _bn_block(x, s1, s2, g, b):
    mean = s1 / N
    var = s2 / N - mean * mean
    scale = g * lax.rsqrt(var + 1e-5)
    shift = b - mean * scale
    return x * scale + shift


def _l1_body(x_ref, s1_ref, s2_ref, g_ref, b_ref, w_ref, as_ref, ad_ref,
             h_ref, sa_ref, da_ref):
    xn = _bn_block(x_ref[...], s1_ref[...], s2_ref[...], g_ref[...], b_ref[...])
    h = jnp.dot(xn, w_ref[...], preferred_element_type=jnp.float32)
    h_ref[...] = h
    sa_ref[0, 0, :] = jnp.sum(h * as_ref[...], axis=1)
    da_ref[0, 0, :] = jnp.sum(h * ad_ref[...], axis=1)


def _l1(x, s1, s2, g, b, w, att_s, att_d):
    d = x.shape[1]
    return pl.pallas_call(
        _l1_body,
        grid=(GRID_M,),
        in_specs=[pl.BlockSpec((BM, d), lambda i: (i, 0)),
                  pl.BlockSpec((1, d), lambda i: (0, 0)),
                  pl.BlockSpec((1, d), lambda i: (0, 0)),
                  pl.BlockSpec((1, d), lambda i: (0, 0)),
                  pl.BlockSpec((1, d), lambda i: (0, 0)),
                  pl.BlockSpec((d, C), lambda i: (0, 0)),
                  pl.BlockSpec((1, C), lambda i: (0, 0)),
                  pl.BlockSpec((1, C), lambda i: (0, 0))],
        out_specs=[pl.BlockSpec((BM, C), lambda i: (i, 0)),
                   pl.BlockSpec((1, 1, BM), lambda i: (i, 0, 0)),
                   pl.BlockSpec((1, 1, BM), lambda i: (i, 0, 0))],
        out_shape=[jax.ShapeDtypeStruct((N, C), jnp.float32),
                   jax.ShapeDtypeStruct((GRID_M, 1, BM), jnp.float32),
                   jax.ShapeDtypeStruct((GRID_M, 1, BM), jnp.float32)],
    )(x, s1, s2, g, b, w, att_s, att_d)


def _l2_body(agg_ref, den_ref, b1_ref, x_ref, s1_ref, s2_ref, g_ref, b_ref,
             w2a_ref, w2b_ref, as_ref, ad_ref, h_ref, sa_ref, da_ref):
    den = den_ref[0, 0, :]
    out1 = agg_ref[...] / (den[:, None] + 1e-16) + b1_ref[...]
    xn = _bn_block(x_ref[...], s1_ref[...], s2_ref[...], g_ref[...], b_ref[...])
    h = (jnp.dot(out1, w2a_ref[...], preferred_element_type=jnp.float32)
         + jnp.dot(xn, w2b_ref[...], preferred_element_type=jnp.float32))
    h_ref[...] = h
    sa_ref[0, 0, :] = jnp.sum(h * as_ref[...], axis=1)
    da_ref[0, 0, :] = jnp.sum(h * ad_ref[...], axis=1)


def _l2(agg, den, b1, x, s1, s2, g, b, w2a, w2b, att_s, att_d):
    d = x.shape[1]
    return pl.pallas_call(
        _l2_body,
        grid=(GRID_M,),
        in_specs=[pl.BlockSpec((BM, C), lambda i: (i, 0)),
                  pl.BlockSpec((1, 1, BM), lambda i: (i, 0, 0)),
                  pl.BlockSpec((1, C), lambda i: (0, 0)),
                  pl.BlockSpec((BM, d), lambda i: (i, 0)),
                  pl.BlockSpec((1, d), lambda i: (0, 0)),
                  pl.BlockSpec((1, d), lambda i: (0, 0)),
                  pl.BlockSpec((1, d), lambda i: (0, 0)),
                  pl.BlockSpec((1, d), lambda i: (0, 0)),
                  pl.BlockSpec((C, C), lambda i: (0, 0)),
                  pl.BlockSpec((d, C), lambda i: (0, 0)),
                  pl.BlockSpec((1, C), lambda i: (0, 0)),
                  pl.BlockSpec((1, C), lambda i: (0, 0))],
        out_specs=[pl.BlockSpec((BM, C), lambda i: (i, 0)),
                   pl.BlockSpec((1, 1, BM), lambda i: (i, 0, 0)),
                   pl.BlockSpec((1, 1, BM), lambda i: (i, 0, 0))],
        out_shape=[jax.ShapeDtypeStruct((N, C), jnp.float32),
                   jax.ShapeDtypeStruct((GRID_M, 1, BM), jnp.float32),
                   jax.ShapeDtypeStruct((GRID_M, 1, BM), jnp.float32)],
    )(agg, den, b1, x, s1, s2, g, b, w2a, w2b, att_s, att_d)


def _pool_body(agg_ref, den_ref, b2_ref, wp_ref, bp_ref, t_ref,
               out_ref, pool_ref, l_ref, a_ref):
    i = pl.program_id(0)
    den = den_ref[0, 0, :]
    out2 = agg_ref[...] / (den[:, None] + 1e-16) + b2_ref[...]
    out_ref[...] = out2
    logits = jnp.dot(out2, wp_ref[...], preferred_element_type=jnp.float32)
    logits = logits + bp_ref[...]
    pool_ref[...] = logits
    lane = lax.broadcasted_iota(jnp.int32, (BM, 128), 1)
    lm = jnp.where(lane < 8, logits, -1e30)
    mx = jnp.max(lm, axis=1)
    lse = mx + jnp.log(jnp.sum(jnp.exp(lm - mx[:, None]), axis=1))
    t = t_ref[0, 0, :]
    tv = jnp.sum(jnp.where(lane == t[:, None], logits, 0.0), axis=1)
    nll = jnp.sum(lse - tv) * (1.0 / N)
    amax = jnp.min(jnp.where(lm == mx[:, None], lane, 127), axis=1)
    corr = jnp.sum((amax == t).astype(jnp.float32)) * (1.0 / N)

    @pl.when(i == 0)
    def _():
        l_ref[0, 0] = nll
        a_ref[0, 0] = corr

    @pl.when(i > 0)
    def _():
        l_ref[0, 0] += nll
        a_ref[0, 0] += corr


def _pooler(agg, den, b2, wp, bp, t3):
    return pl.pallas_call(
        _pool_body,
        grid=(GRID_M,),
        in_specs=[pl.BlockSpec((BM, C), lambda i: (i, 0)),
                  pl.BlockSpec((1, 1, BM), lambda i: (i, 0, 0)),
                  pl.BlockSpec((1, C), lambda i: (0, 0)),
                  pl.BlockSpec((C, 128), lambda i: (0, 0)),
                  pl.BlockSpec((1, 128), lambda i: (0, 0)),
                  pl.BlockSpec((1, 1, BM), lambda i: (i, 0, 0))],
        out_specs=[pl.BlockSpec((BM, C), lambda i: (i, 0)),
                   pl.BlockSpec((BM, 128), lambda i: (i, 0)),
                   pl.BlockSpec(memory_space=pltpu.SMEM),
                   pl.BlockSpec(memory_space=pltpu.SMEM)],
        out_shape=[jax.ShapeDtypeStruct((N, C), jnp.float32),
                   jax.ShapeDtypeStruct((N, 128), jnp.float32),
                   jax.ShapeDtypeStruct((1, 1), jnp.float32),
                   jax.ShapeDtypeStruct((1, 1), jnp.float32)],
    )(agg, den, b2, wp, bp, t3)


def _edge_scalar(a_s, a_d, src, dst):
    mesh = plsc.VectorSubcoreMesh(core_axis_name="c", subcore_axis_name="s",
                                  num_cores=2, num_subcores=16)

    @functools.partial(
        pl.kernel,
        out_type=[jax.ShapeDtypeStruct((32, NCHUNK * CAP), jnp.int32),
                  jax.ShapeDtypeStruct((32, NCHUNK * CAP), jnp.int32),
                  jax.ShapeDtypeStruct((32, NCHUNK * CAP), jnp.float32)],
        mesh=mesh,
        scratch_types=[pltpu.VMEM((NP,), jnp.float32),
                       pltpu.VMEM((NP,), jnp.float32),
                       pltpu.VMEM((EPT3,), jnp.int32),
                       pltpu.VMEM((EPT3,), jnp.int32),
                       pltpu.VMEM((EPT3,), jnp.float32),
                       pltpu.VMEM((EPT3,), jnp.float32),
                       pltpu.VMEM((EPT3,), jnp.float32),
                       pltpu.VMEM((NCHUNK * CAP,), jnp.int32),
                       pltpu.VMEM((NCHUNK * CAP,), jnp.int32),
                       pltpu.VMEM((NCHUNK * CAP,), jnp.float32),
                       pltpu.SemaphoreType.DMA],
        compiler_params=pltpu.CompilerParams(needs_layout_passes=False),
    )
    def k(as_hbm, ad_hbm, src_hbm, dst_hbm,
          bsrc_hbm, bdst_hbm, bw_hbm,
          as_v, ad_v, src_v, dst_v, asg_v, adg_v, w_v,
          bk_src, bk_dst, bk_w, sem):
        cid = lax.axis_index("c")
        sid = lax.axis_index("s")
        wid = sid * 2 + cid
        base = wid * EPT3
        pltpu.sync_copy(as_hbm, as_v)
        pltpu.sync_copy(ad_hbm, ad_v)
        pltpu.sync_copy(src_hbm.at[pl.ds(base, EPT3)], src_v)
        pltpu.sync_copy(dst_hbm.at[pl.ds(base, EPT3)], dst_v)
        pltpu.async_copy(as_hbm.at[src_v], asg_v, sem).wait()
        pltpu.async_copy(ad_hbm.at[dst_v], adg_v, sem).wait()

        zf = jnp.zeros((16,), jnp.float32)
        zi = jnp.zeros((16,), jnp.int32)

        def zb(i, carry):
            ma, md = carry
            ma = jnp.maximum(ma, as_v[pl.ds(i * 16, 16)])
            md = jnp.maximum(md, ad_v[pl.ds(i * 16, 16)])
            return ma, md

        neg = jnp.full((16,), -1e30, jnp.float32)
        ma, md = lax.fori_loop(0, NP // 16, zb, (neg, neg))
        m_s = ma[0]
        m_d = md[0]
        for i in range(1, 16):
            m_s = jnp.maximum(m_s, ma[i])
            m_d = jnp.maximum(m_d, md[i])
        m_ub = jnp.maximum(m_s + m_d, 0.0)

        def eb(g, m):
            dv = dst_v[pl.ds(g * 16, 16)]
            z = asg_v[pl.ds(g * 16, 16)] + adg_v[pl.ds(g * 16, 16)]
            e = jnp.maximum(z, 0.2 * z)
            w = jnp.exp(e - m)
            ge = base + g * 16 + lax.iota(jnp.int32, 16)
            w = jnp.where(ge < E_TOT, w, 0.0)
            w_v[pl.ds(g * 16, 16)] = w
            return m

        lax.fori_loop(0, EPT3 // 16, eb, m_ub)
        _ = dst_v

        def pf(i, _):
            cc = i // (CAP // 16)
            j = i % (CAP // 16)
            bk_src[pl.ds(cc * CAP + j * 16, 16)] = zi
            bk_dst[pl.ds(cc * CAP + j * 16, 16)] = zi + cc * RCH
            bk_w[pl.ds(cc * CAP + j * 16, 16)] = zf
            return 0

        lax.fori_loop(0, NCHUNK * (CAP // 16), pf, 0)

        def bb(g, cnts):
            sv = src_v[pl.ds(g * 16, 16)]
            dv = dst_v[pl.ds(g * 16, 16)]
            wv = w_v[pl.ds(g * 16, 16)]
            out = []
            for cc in range(NCHUNK):
                cnt = jnp.minimum(cnts[cc], CAP - 16)
                msk = (dv >= cc * RCH) & (dv < (cc + 1) * RCH)
                o = cc * CAP + cnt
                plsc.store_compressed(bk_src.at[pl.ds(o, 16)], sv, mask=msk)
                plsc.store_compressed(bk_dst.at[pl.ds(o, 16)], dv, mask=msk)
                plsc.store_compressed(bk_w.at[pl.ds(o, 16)], wv, mask=msk)
                pc = plsc.all_reduce_population_count(msk)
                out.append(cnt + pc[0])
            return tuple(out)

        lax.fori_loop(0, EPT3 // 16, bb, (0, 0, 0, 0, 0, 0, 0, 0))

        pltpu.sync_copy(bk_src, bsrc_hbm.at[wid])
        pltpu.sync_copy(bk_dst, bdst_hbm.at[wid])
        pltpu.sync_copy(bk_w, bw_hbm.at[wid])

    return k(a_s, a_d, src, dst)


CLCAP = 2064
WIN = RCH // 16


def _edge_rows(h, bsrc, bdst, bw):
    mesh = plsc.VectorSubcoreMesh(core_axis_name="c", subcore_axis_name="s",
                                  num_cores=2, num_subcores=16)

    @functools.partial(
        pl.kernel,
        out_type=[jax.ShapeDtypeStruct((NCHUNK * RCH, C), jnp.float32),
                  jax.ShapeDtypeStruct((NCHUNK * RCH, 16), jnp.float32)],
        mesh=mesh,
        scratch_types=[pltpu.VMEM((WIN, C), jnp.float32),
                       pltpu.VMEM((WIN, 16), jnp.float32),
                       pltpu.VMEM((CAP,), jnp.int32),
                       pltpu.VMEM((CAP,), jnp.int32),
                       pltpu.VMEM((CAP,), jnp.float32),
                       pltpu.VMEM((CLCAP,), jnp.int32),
                       pltpu.VMEM((CLCAP,), jnp.int32),
                       pltpu.VMEM((CLCAP,), jnp.float32),
                       pltpu.VMEM((16, C), jnp.float32),
                       pltpu.SemaphoreType.DMA],
        compiler_params=pltpu.CompilerParams(needs_layout_passes=False),
    )
    def k(h_hbm, bsrc_hbm, bdst_hbm, bw_hbm, agg_hbm, dena_hbm,
          acc, dacc, st_src, st_dst, st_w, c_src, c_dst, c_w, rowbuf, sem):
        cid = lax.axis_index("c")
        sid = lax.axis_index("s")
        zf = jnp.zeros((16,), jnp.float32)
        zi = jnp.zeros((16,), jnp.int32)

        def chunk(q, _):
            qc = cid * (NCHUNK // 2) + q
            wlo = qc * RCH + sid * WIN

            def zrow(i, _):
                acc[i // CG, pl.ds((i % CG) * 16, 16)] = zf
                return 0

            lax.fori_loop(0, WIN * CG, zrow, 0)

            def zden(i, _):
                dacc[i, pl.ds(0, 16)] = zf
                return 0

            lax.fori_loop(0, WIN, zden, 0)

            def st_loop(st, cnt):
                pltpu.sync_copy(bsrc_hbm.at[st, pl.ds(qc * CAP, CAP)], st_src)
                pltpu.sync_copy(bdst_hbm.at[st, pl.ds(qc * CAP, CAP)], st_dst)
                pltpu.sync_copy(bw_hbm.at[st, pl.ds(qc * CAP, CAP)], st_w)

                def cg_loop(g, cnt):
                    dv = st_dst[pl.ds(g * 16, 16)]
                    wv0 = st_w[pl.ds(g * 16, 16)]
                    msk = (dv >= wlo) & (dv < wlo + WIN) & (wv0 > 0.0)
                    cl = jnp.minimum(cnt, CLCAP - 16)
                    plsc.store_compressed(c_src.at[pl.ds(cl, 16)],
                                          st_src[pl.ds(g * 16, 16)],
                                          mask=msk)
                    plsc.store_compressed(c_dst.at[pl.ds(cl, 16)], dv,
                                          mask=msk)
                    plsc.store_compressed(c_w.at[pl.ds(cl, 16)],
                                          st_w[pl.ds(g * 16, 16)],
                                          mask=msk)
                    pc = plsc.all_reduce_population_count(msk)
                    return cl + pc[0]

                return lax.fori_loop(0, CAP // 16, cg_loop, cnt)

            cnt = lax.fori_loop(0, 32, st_loop, 0)
            c_src[pl.ds(cnt, 16)] = zi
            c_dst[pl.ds(cnt, 16)] = zi + wlo
            c_w[pl.ds(cnt, 16)] = zf
            nsub = (cnt + 15) // 16

            def proc(t, _):
                pltpu.async_copy(
                    h_hbm.at[c_src.at[pl.ds(t * 16, 16)]], rowbuf,
                    sem).wait()
                dv = c_dst[pl.ds(t * 16, 16)]
                wv = c_w[pl.ds(t * 16, 16)]
                dl = dv - wlo
                for e2 in range(16):
                    d = dl[e2]
                    ws = wv[e2]
                    plsc.addupdate(dacc.at[d, pl.ds(0, 16)],
                                   jnp.zeros((16,), jnp.float32) + ws)
                    for gg in range(CG):
                        plsc.addupdate(acc.at[d, pl.ds(gg * 16, 16)],
                                       rowbuf[e2, pl.ds(gg * 16, 16)] * ws)
                return 0

            lax.fori_loop(0, nsub, proc, 0)
            pltpu.sync_copy(acc, agg_hbm.at[pl.ds(wlo, WIN)])
            pltpu.sync_copy(dacc, dena_hbm.at[pl.ds(wlo, WIN)])
            return 0

        lax.fori_loop(0, NCHUNK // 2, chunk, 0)

    return k(h, bsrc, bdst, bw)


def kernel(x, bn_gamma, bn_beta, W1, att_src1, att_dst1, b1,
           W2, att_src2, att_dst2, b2, Wp, bp, edge_index, target):
    loops = jnp.arange(N, dtype=jnp.int32)
    padz = jnp.zeros((EP - E_TOT,), jnp.int32)
    src = jnp.concatenate([edge_index[0], loops, padz])
    dst = jnp.concatenate([edge_index[1], loops, padz])

    s1, s2 = _stats(x)
    g2 = bn_gamma[None, :]
    be2 = bn_beta[None, :]

    h1, as1, ad1 = _l1(x, s1, s2, g2, be2, W1,
                       att_src1[None, :], att_dst1[None, :])
    as1p = jnp.pad(as1.reshape(-1), (0, NP - N))
    ad1p = jnp.pad(ad1.reshape(-1), (0, NP - N))
    bs1, bd1, bw1 = _edge_scalar(as1p, ad1p, src, dst)
    agg1, dena1 = _edge_rows(h1, bs1, bd1, bw1)
    den1t = dena1[:N, 0].reshape(GRID_M, 1, BM)

    h2, as2, ad2 = _l2(agg1, den1t, b1[None, :], x, s1, s2, g2, be2,
                       W2[:C], W2[C:], att_src2[None, :], att_dst2[None, :])
    as2p = jnp.pad(as2.reshape(-1), (0, NP - N))
    ad2p = jnp.pad(ad2.reshape(-1), (0, NP - N))
    bs2, bd2, bw2 = _edge_scalar(as2p, ad2p, src, dst)
    agg2, dena2 = _edge_rows(h2, bs2, bd2, bw2)
    den2t = dena2[:N, 0].reshape(GRID_M, 1, BM)

    wp_pad = jnp.pad(Wp, ((0, 0), (0, 120)))
    bp_pad = jnp.pad(bp, (0, 120))[None, :]
    t3 = target.reshape(GRID_M, 1, BM)
    out2, pool, lsum, asum = _pooler(agg2, den2t, b2[None, :], wp_pad,
                                     bp_pad, t3)
    return out2, pool[:, :8], lsum.reshape(()), asum.reshape(())
